# Initial kernel scaffold; baseline (speedup 1.0000x reference)
import jax
import jax.numpy as jnp
from jax.experimental import pallas as pl


def kernel(x, edge_index, edge_weight, W1, b1, W2, b2):
    raise NotImplementedError("write your pallas kernel here")


# speedup vs baseline: 6.6971x; 6.6971x over previous
import functools

import jax
import jax.numpy as jnp
from jax import lax
from jax.experimental import pallas as pl
from jax.experimental.pallas import tpu as pltpu
from jax.experimental.pallas import tpu_sc as plsc

N_NODES = 10000
N_EDGES = 320000
FDIM = 128
CLIPV = 100000.0

NP_ = 10240
EP_ = 327680
NC = 2
NS = 16
NW = NC * NS
EPW = EP_ // NW
BLK = 128
NBLK = EPW // BLK
STRIPE = NP_ // NS

_mesh = plsc.VectorSubcoreMesh(core_axis_name="c", subcore_axis_name="s",
                               num_cores=NC, num_subcores=NS)
_sc_params = pltpu.CompilerParams(needs_layout_passes=False)


@functools.partial(
    pl.kernel,
    out_type=jax.ShapeDtypeStruct((NW, NP_), jnp.float32),
    mesh=_mesh,
    scratch_types=[
        pltpu.VMEM((NP_,), jnp.float32),
        pltpu.VMEM((EPW,), jnp.int32),
        pltpu.VMEM((EPW,), jnp.float32),
    ],
    compiler_params=_sc_params,
)
def _deg_kernel(dst_hbm, ew_hbm, out_hbm, deg_v, dst_v, ew_v):
    c = lax.axis_index("c")
    s = lax.axis_index("s")
    wid = c * NS + s
    base = wid * EPW
    pltpu.sync_copy(dst_hbm.at[pl.ds(base, EPW)], dst_v)
    pltpu.sync_copy(ew_hbm.at[pl.ds(base, EPW)], ew_v)

    def zero_body(i, _):
        deg_v[pl.ds(i * 16, 16)] = jnp.zeros((16,), jnp.float32)
        return _
    lax.fori_loop(0, NP_ // 16, zero_body, None)

    def acc_body(i, _):
        idx = dst_v[pl.ds(i * 16, 16)]
        w = ew_v[pl.ds(i * 16, 16)]
        plsc.addupdate_scatter(deg_v, [idx], w)
        return _
    lax.fori_loop(0, EPW // 16, acc_body, None)

    pltpu.sync_copy(deg_v, out_hbm.at[wid])


@functools.partial(
    pl.kernel,
    out_type=jax.ShapeDtypeStruct((NC, NP_, FDIM), jnp.float32),
    mesh=_mesh,
    scratch_types=[
        pltpu.VMEM((NP_,), jnp.float32),
        pltpu.VMEM((BLK,), jnp.int32),
        pltpu.VMEM((BLK,), jnp.int32),
        pltpu.VMEM((BLK,), jnp.float32),
        pltpu.VMEM((BLK, FDIM), jnp.float32),
        pltpu.VMEM_SHARED((NP_, FDIM), jnp.float32),
        pltpu.SemaphoreType.DMA,
    ],
    compiler_params=_sc_params,
)
def _agg_kernel(tbl_hbm, src_hbm, dst_hbm, ew_hbm, dis_hbm, out_hbm,
                dis_v, src_v, dst_v, ew_v, rows_v, acc_sh, sem):
    c = lax.axis_index("c")
    s = lax.axis_index("s")
    wid = c * NS + s
    base = wid * EPW

    pltpu.sync_copy(dis_hbm, dis_v)

    def zrow_body(i, _):
        for j in range(FDIM // 16):
            rows_v[i, pl.ds(j * 16, 16)] = jnp.zeros((16,), jnp.float32)
        return _
    lax.fori_loop(0, BLK, zrow_body, None)
    for k in range(STRIPE // BLK):
        pltpu.sync_copy(rows_v, acc_sh.at[pl.ds(s * STRIPE + k * BLK, BLK)])
    plsc.subcore_barrier()

    def blk_body(b, _):
        eb = base + b * BLK
        pltpu.sync_copy(src_hbm.at[pl.ds(eb, BLK)], src_v)
        pltpu.sync_copy(dst_hbm.at[pl.ds(eb, BLK)], dst_v)
        pltpu.sync_copy(ew_hbm.at[pl.ds(eb, BLK)], ew_v)
        pltpu.async_copy(tbl_hbm.at[src_v], rows_v, sem).wait()

        def grp_body(g, _g):
            dvec = dst_v[pl.ds(g * 16, 16)]
            disg = plsc.load_gather(dis_v, [dvec])
            nrm = ew_v[pl.ds(g * 16, 16)] * disg
            for l in range(16):
                e = g * 16 + l
                sc = lax.broadcast_in_dim(nrm[l], (16,), ())
                for j in range(FDIM // 16):
                    rows_v[e, pl.ds(j * 16, 16)] = (
                        rows_v[e, pl.ds(j * 16, 16)] * sc)
            return _g
        lax.fori_loop(0, BLK // 16, grp_body, None)

        pltpu.sync_copy(rows_v, acc_sh.at[dst_v], add=True)
        return _
    lax.fori_loop(0, NBLK, blk_body, None)

    plsc.subcore_barrier()
    pltpu.sync_copy(acc_sh.at[pl.ds(s * STRIPE, STRIPE)],
                    out_hbm.at[c, pl.ds(s * STRIPE, STRIPE)])


def _n2n(v):
    v = jnp.where(jnp.isnan(v), 0.0, v)
    v = jnp.where(v == jnp.inf, CLIPV, v)
    v = jnp.where(v == -jnp.inf, -CLIPV, v)
    return v


def _tc_dis_body(degp_ref, o_ref):
    deg = jnp.sum(degp_ref[...], axis=0, keepdims=True)
    row = lax.broadcasted_iota(jnp.int32, (1, NP_), 1)
    deg = deg + jnp.where(row < N_NODES, 1.0, 0.0)
    o_ref[...] = jnp.where(deg > 0, lax.rsqrt(deg), 0.0)


def _tc_xws_body(x_ref, w_ref, dis_ref, o_ref):
    xw = jnp.dot(x_ref[...], w_ref[...],
                 preferred_element_type=jnp.float32,
                 precision=lax.Precision.HIGHEST)
    o_ref[...] = xw * dis_ref[...]


def _tc_mid_body(p_ref, xws_ref, dis_ref, b_ref, w_ref, o_ref):
    h = p_ref[0] + p_ref[1] + xws_ref[...] * dis_ref[...] + b_ref[...]
    h = jax.nn.relu(_n2n(h))
    xw = jnp.dot(h, w_ref[...], preferred_element_type=jnp.float32,
                 precision=lax.Precision.HIGHEST)
    o_ref[...] = xw * dis_ref[...]


def _tc_post_body(p_ref, xws_ref, dis_ref, b_ref, o_ref):
    h = p_ref[0] + p_ref[1] + xws_ref[...] * dis_ref[...] + b_ref[...]
    h = jax.nn.relu(_n2n(h))
    row = lax.broadcasted_iota(jnp.int32, (NP_, 1), 0)
    h = jnp.where(row < N_NODES, h, 0.0)
    g = jnp.sum(h, axis=0, keepdims=True) * (1.0 / N_NODES)
    o_ref[...] = _n2n(g)


def kernel(x, edge_index, edge_weight, W1, b1, W2, b2):
    src = edge_index[0].astype(jnp.int32)
    dst = edge_index[1].astype(jnp.int32)
    pad_e = EP_ - N_EDGES
    src_p = jnp.concatenate([src, jnp.full((pad_e,), NP_ - 1, jnp.int32)])
    dst_p = jnp.concatenate([dst, jnp.full((pad_e,), NP_ - 1, jnp.int32)])
    ew_p = jnp.concatenate([edge_weight.astype(jnp.float32),
                            jnp.zeros((pad_e,), jnp.float32)])
    x_p = jnp.pad(x, ((0, NP_ - N_NODES), (0, 0)))
    b1r = b1.reshape(1, FDIM)
    b2r = b2.reshape(1, FDIM)

    degp = _deg_kernel(dst_p, ew_p)

    dis2d = pl.pallas_call(
        _tc_dis_body,
        out_shape=jax.ShapeDtypeStruct((1, NP_), jnp.float32),
    )(degp)
    dis_flat = dis2d.reshape(NP_)
    dis_col = dis2d.reshape(NP_, 1)

    xws1 = pl.pallas_call(
        _tc_xws_body,
        out_shape=jax.ShapeDtypeStruct((NP_, FDIM), jnp.float32),
    )(x_p, W1, dis_col)

    p1 = _agg_kernel(xws1, src_p, dst_p, ew_p, dis_flat)

    xws2 = pl.pallas_call(
        _tc_mid_body,
        out_shape=jax.ShapeDtypeStruct((NP_, FDIM), jnp.float32),
    )(p1, xws1, dis_col, b1r, W2)

    p2 = _agg_kernel(xws2, src_p, dst_p, ew_p, dis_flat)

    g2d = pl.pallas_call(
        _tc_post_body,
        out_shape=jax.ShapeDtypeStruct((1, FDIM), jnp.float32),
    )(p2, xws2, dis_col, b2r)

    return g2d.reshape(FDIM)


# speedup vs baseline: 9.1834x; 1.3713x over previous
import functools

import jax
import jax.numpy as jnp
from jax import lax
from jax.experimental import pallas as pl
from jax.experimental.pallas import tpu as pltpu
from jax.experimental.pallas import tpu_sc as plsc

N_NODES = 10000
N_EDGES = 320000
FDIM = 128
CLIPV = 100000.0

NP_ = 10240
EP_ = 327680
NC = 2
NS = 16
NW = NC * NS
EPW = EP_ // NW
BLK = 128
NBLK = EPW // BLK
STRIPE = NP_ // NS

_mesh = plsc.VectorSubcoreMesh(core_axis_name="c", subcore_axis_name="s",
                               num_cores=NC, num_subcores=NS)
_sc_params = pltpu.CompilerParams(needs_layout_passes=False)


@functools.partial(
    pl.kernel,
    out_type=jax.ShapeDtypeStruct((NW, NP_), jnp.float32),
    mesh=_mesh,
    scratch_types=[
        pltpu.VMEM((NP_,), jnp.float32),
        pltpu.VMEM((EPW,), jnp.int32),
        pltpu.VMEM((EPW,), jnp.float32),
    ],
    compiler_params=_sc_params,
)
def _deg_kernel(dst_hbm, ew_hbm, out_hbm, deg_v, dst_v, ew_v):
    c = lax.axis_index("c")
    s = lax.axis_index("s")
    wid = c * NS + s
    base = wid * EPW
    pltpu.sync_copy(dst_hbm.at[pl.ds(base, EPW)], dst_v)
    pltpu.sync_copy(ew_hbm.at[pl.ds(base, EPW)], ew_v)

    def zero_body(i, _):
        deg_v[pl.ds(i * 16, 16)] = jnp.zeros((16,), jnp.float32)
        return _
    lax.fori_loop(0, NP_ // 16, zero_body, None)

    def acc_body(i, _):
        idx = dst_v[pl.ds(i * 16, 16)]
        w = ew_v[pl.ds(i * 16, 16)]
        plsc.addupdate_scatter(deg_v, [idx], w)
        return _
    lax.fori_loop(0, EPW // 16, acc_body, None)

    pltpu.sync_copy(deg_v, out_hbm.at[wid])


@functools.partial(
    pl.kernel,
    out_type=jax.ShapeDtypeStruct((NC, NP_, FDIM), jnp.float32),
    mesh=_mesh,
    scratch_types=(
        [pltpu.VMEM((NP_,), jnp.float32)]
        + [pltpu.VMEM((BLK,), jnp.int32) for _ in range(8)]
        + [pltpu.VMEM((BLK,), jnp.float32) for _ in range(4)]
        + [pltpu.VMEM((BLK, FDIM), jnp.float32) for _ in range(2)]
        + [pltpu.VMEM_SHARED((NP_, FDIM), jnp.float32)]
        + [pltpu.SemaphoreType.DMA for _ in range(8)]
    ),
    compiler_params=_sc_params,
)
def _agg_kernel(tbl_hbm, src_hbm, dst_hbm, ew_hbm, dis_hbm, out_hbm,
                dis_v, src0, src1, src2, src3, dst0, dst1, dst2, dst3,
                ew0, ew1, ew2, ew3, rowsA, rowsB, acc_sh,
                isem0, isem1, isem2, isem3, gsemA, gsemB, ssemA, ssemB):
    c = lax.axis_index("c")
    s = lax.axis_index("s")
    wid = c * NS + s
    base = wid * EPW

    idx_sets = ((src0, dst0, ew0, isem0), (src1, dst1, ew1, isem1),
                (src2, dst2, ew2, isem2), (src3, dst3, ew3, isem3))
    row_sets = ((rowsA, gsemA, ssemA), (rowsB, gsemB, ssemB))

    pltpu.sync_copy(dis_hbm, dis_v)

    def zrow_body(i, _):
        for j in range(FDIM // 16):
            rowsA[i, pl.ds(j * 16, 16)] = jnp.zeros((16,), jnp.float32)
        return _
    lax.fori_loop(0, BLK, zrow_body, None)
    for k in range(STRIPE // BLK):
        pltpu.sync_copy(rowsA, acc_sh.at[pl.ds(s * STRIPE + k * BLK, BLK)])
    plsc.subcore_barrier()

    def start_idx(b, iset):
        eb = base + b * BLK
        pltpu.async_copy(src_hbm.at[pl.ds(eb, BLK)], iset[0], iset[3])
        pltpu.async_copy(dst_hbm.at[pl.ds(eb, BLK)], iset[1], iset[3])
        pltpu.async_copy(ew_hbm.at[pl.ds(eb, BLK)], iset[2], iset[3])

    def wait_idx(iset):
        pltpu.make_async_copy(src_hbm.at[pl.ds(0, BLK)], iset[0],
                              iset[3]).wait()
        pltpu.make_async_copy(dst_hbm.at[pl.ds(0, BLK)], iset[1],
                              iset[3]).wait()
        pltpu.make_async_copy(ew_hbm.at[pl.ds(0, BLK)], iset[2],
                              iset[3]).wait()

    def start_gather(iset, rset):
        pltpu.async_copy(tbl_hbm.at[iset[0]], rset[0], rset[1])

    def wait_gather(rset):
        pltpu.make_async_copy(tbl_hbm.at[pl.ds(0, BLK)], rset[0],
                              rset[1]).wait()

    def scale(rset, iset):
        rows_v, dst_v, ew_v = rset[0], iset[1], iset[2]

        def grp_body(g, _g):
            dvec = dst_v[pl.ds(g * 16, 16)]
            disg = plsc.load_gather(dis_v, [dvec])
            nrm = ew_v[pl.ds(g * 16, 16)] * disg
            for l in range(16):
                e = g * 16 + l
                sc = lax.broadcast_in_dim(nrm[l], (16,), ())
                for j in range(FDIM // 16):
                    rows_v[e, pl.ds(j * 16, 16)] = (
                        rows_v[e, pl.ds(j * 16, 16)] * sc)
            return _g
        lax.fori_loop(0, BLK // 16, grp_body, None)

    def scatter(rset, iset):
        pltpu.async_copy(rset[0], acc_sh.at[iset[1]], rset[2], add=True)
        pltpu.make_async_copy(tbl_hbm.at[pl.ds(0, BLK)], rset[0],
                              rset[2]).wait()

    def visit(b, q, pf_idx, pf_gather):
        iset = idx_sets[q]
        rset = row_sets[q % 2]
        wait_gather(rset)
        scale(rset, iset)
        scatter(rset, iset)
        if pf_idx:
            start_idx(b + 4, iset)
        if pf_gather:
            nset = idx_sets[(q + 2) % 4]
            wait_idx(nset)
            start_gather(nset, rset)

    for q in range(4):
        start_idx(q, idx_sets[q])
    wait_idx(idx_sets[0])
    start_gather(idx_sets[0], row_sets[0])
    wait_idx(idx_sets[1])
    start_gather(idx_sets[1], row_sets[1])

    def quad_body(k, _):
        b = k * 4
        for q in range(4):
            visit(b + q, q, True, True)
        return _
    lax.fori_loop(0, NBLK // 4 - 1, quad_body, None)

    bq = NBLK - 4
    for q in range(4):
        visit(bq + q, q, False, q < 2)

    plsc.subcore_barrier()
    pltpu.sync_copy(acc_sh.at[pl.ds(s * STRIPE, STRIPE)],
                    out_hbm.at[c, pl.ds(s * STRIPE, STRIPE)])


def _n2n(v):
    v = jnp.where(jnp.isnan(v), 0.0, v)
    v = jnp.where(v == jnp.inf, CLIPV, v)
    v = jnp.where(v == -jnp.inf, -CLIPV, v)
    return v


def _tc_dis_body(degp_ref, o_ref):
    deg = jnp.sum(degp_ref[...], axis=0, keepdims=True)
    row = lax.broadcasted_iota(jnp.int32, (1, NP_), 1)
    deg = deg + jnp.where(row < N_NODES, 1.0, 0.0)
    o_ref[...] = jnp.where(deg > 0, lax.rsqrt(deg), 0.0)


def _tc_xws_body(x_ref, w_ref, dis_ref, o_ref):
    xw = jnp.dot(x_ref[...], w_ref[...],
                 preferred_element_type=jnp.float32,
                 precision=lax.Precision.HIGHEST)
    o_ref[...] = xw * dis_ref[...]


def _tc_mid_body(p_ref, xws_ref, dis_ref, b_ref, w_ref, o_ref):
    h = p_ref[0] + p_ref[1] + xws_ref[...] * dis_ref[...] + b_ref[...]
    h = jax.nn.relu(_n2n(h))
    xw = jnp.dot(h, w_ref[...], preferred_element_type=jnp.float32,
                 precision=lax.Precision.HIGHEST)
    o_ref[...] = xw * dis_ref[...]


def _tc_post_body(p_ref, xws_ref, dis_ref, b_ref, o_ref):
    h = p_ref[0] + p_ref[1] + xws_ref[...] * dis_ref[...] + b_ref[...]
    h = jax.nn.relu(_n2n(h))
    row = lax.broadcasted_iota(jnp.int32, (NP_, 1), 0)
    h = jnp.where(row < N_NODES, h, 0.0)
    g = jnp.sum(h, axis=0, keepdims=True) * (1.0 / N_NODES)
    o_ref[...] = _n2n(g)


def kernel(x, edge_index, edge_weight, W1, b1, W2, b2):
    src = edge_index[0].astype(jnp.int32)
    dst = edge_index[1].astype(jnp.int32)
    pad_e = EP_ - N_EDGES
    src_p = jnp.concatenate([src, jnp.full((pad_e,), NP_ - 1, jnp.int32)])
    dst_p = jnp.concatenate([dst, jnp.full((pad_e,), NP_ - 1, jnp.int32)])
    ew_p = jnp.concatenate([edge_weight.astype(jnp.float32),
                            jnp.zeros((pad_e,), jnp.float32)])
    x_p = jnp.pad(x, ((0, NP_ - N_NODES), (0, 0)))
    b1r = b1.reshape(1, FDIM)
    b2r = b2.reshape(1, FDIM)

    degp = _deg_kernel(dst_p, ew_p)

    dis2d = pl.pallas_call(
        _tc_dis_body,
        out_shape=jax.ShapeDtypeStruct((1, NP_), jnp.float32),
    )(degp)
    dis_flat = dis2d.reshape(NP_)
    dis_col = dis2d.reshape(NP_, 1)

    xws1 = pl.pallas_call(
        _tc_xws_body,
        out_shape=jax.ShapeDtypeStruct((NP_, FDIM), jnp.float32),
    )(x_p, W1, dis_col)

    p1 = _agg_kernel(xws1, src_p, dst_p, ew_p, dis_flat)

    xws2 = pl.pallas_call(
        _tc_mid_body,
        out_shape=jax.ShapeDtypeStruct((NP_, FDIM), jnp.float32),
    )(p1, xws1, dis_col, b1r, W2)

    p2 = _agg_kernel(xws2, src_p, dst_p, ew_p, dis_flat)

    g2d = pl.pallas_call(
        _tc_post_body,
        out_shape=jax.ShapeDtypeStruct((1, FDIM), jnp.float32),
    )(p2, xws2, dis_col, b2r)

    return g2d.reshape(FDIM)


# speedup vs baseline: 9.9837x; 1.0871x over previous
import functools

import jax
import jax.numpy as jnp
from jax import lax
from jax.experimental import pallas as pl
from jax.experimental.pallas import tpu as pltpu
from jax.experimental.pallas import tpu_sc as plsc

N_NODES = 10000
N_EDGES = 320000
FDIM = 128
CLIPV = 100000.0

NP_ = 10240
EP_ = 327680
NC = 2
NS = 16
NW = NC * NS
EPW = EP_ // NW
BLK = 80
NBLK = EPW // BLK
STRIPE = NP_ // NS

_mesh = plsc.VectorSubcoreMesh(core_axis_name="c", subcore_axis_name="s",
                               num_cores=NC, num_subcores=NS)
_sc_params = pltpu.CompilerParams(needs_layout_passes=False)


@functools.partial(
    pl.kernel,
    out_type=jax.ShapeDtypeStruct((NW, NP_), jnp.float32),
    mesh=_mesh,
    scratch_types=[
        pltpu.VMEM((NP_,), jnp.float32),
        pltpu.VMEM((EPW,), jnp.int32),
        pltpu.VMEM((EPW,), jnp.float32),
    ],
    compiler_params=_sc_params,
)
def _deg_kernel(dst_hbm, ew_hbm, out_hbm, deg_v, dst_v, ew_v):
    c = lax.axis_index("c")
    s = lax.axis_index("s")
    wid = c * NS + s
    base = wid * EPW
    pltpu.sync_copy(dst_hbm.at[pl.ds(base, EPW)], dst_v)
    pltpu.sync_copy(ew_hbm.at[pl.ds(base, EPW)], ew_v)

    def zero_body(i, _):
        deg_v[pl.ds(i * 16, 16)] = jnp.zeros((16,), jnp.float32)
        return _
    lax.fori_loop(0, NP_ // 16, zero_body, None)

    def acc_body(i, _):
        idx = dst_v[pl.ds(i * 16, 16)]
        w = ew_v[pl.ds(i * 16, 16)]
        plsc.addupdate_scatter(deg_v, [idx], w)
        return _
    lax.fori_loop(0, EPW // 16, acc_body, None)

    pltpu.sync_copy(deg_v, out_hbm.at[wid])


@functools.partial(
    pl.kernel,
    out_type=jax.ShapeDtypeStruct((EP_,), jnp.float32),
    mesh=_mesh,
    scratch_types=[
        pltpu.VMEM((NP_,), jnp.float32),
        pltpu.VMEM((EPW,), jnp.int32),
        pltpu.VMEM((EPW,), jnp.float32),
        pltpu.VMEM((EPW,), jnp.float32),
    ],
    compiler_params=_sc_params,
)
def _nrm_kernel(dst_hbm, ew_hbm, dis_hbm, out_hbm, dis_v, dst_v, ew_v, nrm_v):
    c = lax.axis_index("c")
    s = lax.axis_index("s")
    wid = c * NS + s
    base = wid * EPW
    pltpu.sync_copy(dis_hbm, dis_v)
    pltpu.sync_copy(dst_hbm.at[pl.ds(base, EPW)], dst_v)
    pltpu.sync_copy(ew_hbm.at[pl.ds(base, EPW)], ew_v)

    def body(i, _):
        idx = dst_v[pl.ds(i * 16, 16)]
        disg = plsc.load_gather(dis_v, [idx])
        nrm_v[pl.ds(i * 16, 16)] = ew_v[pl.ds(i * 16, 16)] * disg
        return _
    lax.fori_loop(0, EPW // 16, body, None)
    pltpu.sync_copy(nrm_v, out_hbm.at[pl.ds(base, EPW)])


@functools.partial(
    pl.kernel,
    out_type=jax.ShapeDtypeStruct((NC, NP_, FDIM), jnp.float32),
    mesh=_mesh,
    scratch_types=(
        [pltpu.VMEM((BLK,), jnp.int32) for _ in range(8)]
        + [pltpu.VMEM((BLK,), jnp.float32) for _ in range(4)]
        + [pltpu.VMEM((BLK, FDIM), jnp.float32) for _ in range(4)]
        + [pltpu.VMEM_SHARED((NP_, FDIM), jnp.float32)]
        + [pltpu.SemaphoreType.DMA for _ in range(12)]
    ),
    compiler_params=_sc_params,
)
def _agg_kernel(tbl_hbm, src_hbm, dst_hbm, nrm_hbm, out_hbm,
                src0, src1, src2, src3, dst0, dst1, dst2, dst3,
                nr0, nr1, nr2, nr3, rows0, rows1, rows2, rows3, acc_sh,
                isem0, isem1, isem2, isem3, gsem0, gsem1, gsem2, gsem3,
                ssem0, ssem1, ssem2, ssem3):
    c = lax.axis_index("c")
    s = lax.axis_index("s")
    wid = c * NS + s
    base = wid * EPW

    idx_sets = ((src0, dst0, nr0, isem0), (src1, dst1, nr1, isem1),
                (src2, dst2, nr2, isem2), (src3, dst3, nr3, isem3))
    row_sets = ((rows0, gsem0, ssem0), (rows1, gsem1, ssem1),
                (rows2, gsem2, ssem2), (rows3, gsem3, ssem3))

    def zrow_body(i, _):
        for j in range(FDIM // 16):
            rows0[i, pl.ds(j * 16, 16)] = jnp.zeros((16,), jnp.float32)
        return _
    lax.fori_loop(0, BLK, zrow_body, None)
    for k in range(STRIPE // BLK):
        pltpu.sync_copy(rows0, acc_sh.at[pl.ds(s * STRIPE + k * BLK, BLK)])
    plsc.subcore_barrier()

    def start_idx(b, iset):
        eb = base + b * BLK
        pltpu.async_copy(src_hbm.at[pl.ds(eb, BLK)], iset[0], iset[3])
        pltpu.async_copy(dst_hbm.at[pl.ds(eb, BLK)], iset[1], iset[3])
        pltpu.async_copy(nrm_hbm.at[pl.ds(eb, BLK)], iset[2], iset[3])

    def wait_idx(iset):
        pltpu.make_async_copy(src_hbm.at[pl.ds(0, BLK)], iset[0],
                              iset[3]).wait()
        pltpu.make_async_copy(dst_hbm.at[pl.ds(0, BLK)], iset[1],
                              iset[3]).wait()
        pltpu.make_async_copy(nrm_hbm.at[pl.ds(0, BLK)], iset[2],
                              iset[3]).wait()

    def start_gather(iset, rset):
        pltpu.async_copy(tbl_hbm.at[iset[0]], rset[0], rset[1])

    def wait_gather(rset):
        pltpu.make_async_copy(tbl_hbm.at[pl.ds(0, BLK)], rset[0],
                              rset[1]).wait()

    def scale(rset, iset):
        rows_v, nrm_v = rset[0], iset[2]

        def grp_body(g, _g):
            nrm = nrm_v[pl.ds(g * 16, 16)]
            for l in range(16):
                e = g * 16 + l
                sc = lax.broadcast_in_dim(nrm[l], (16,), ())
                for j in range(FDIM // 16):
                    rows_v[e, pl.ds(j * 16, 16)] = (
                        rows_v[e, pl.ds(j * 16, 16)] * sc)
            return _g
        lax.fori_loop(0, BLK // 16, grp_body, None)

    def scatter(rset, iset):
        pltpu.async_copy(rset[0], acc_sh.at[iset[1]], rset[2], add=True)
        pltpu.make_async_copy(tbl_hbm.at[pl.ds(0, BLK)], rset[0],
                              rset[2]).wait()

    def visit(b, q, pf_idx, pf_gather):
        rset = row_sets[q]
        iset = idx_sets[q]
        wait_gather(rset)
        scale(rset, iset)
        scatter(rset, iset)
        if pf_idx:
            start_idx(b + 4, iset)
        if pf_gather:
            nq = (q + 3) % 4
            wait_idx(idx_sets[nq])
            start_gather(idx_sets[nq], row_sets[nq])

    for q in range(4):
        start_idx(q, idx_sets[q])
    for q in range(3):
        wait_idx(idx_sets[q])
        start_gather(idx_sets[q], row_sets[q])

    def quad_body(k, _):
        b = k * 4
        for q in range(4):
            visit(b + q, q, True, True)
        return _
    lax.fori_loop(0, NBLK // 4 - 1, quad_body, None)

    bq = NBLK - 4
    visit(bq + 0, 0, False, True)
    visit(bq + 1, 1, False, False)
    visit(bq + 2, 2, False, False)
    visit(bq + 3, 3, False, False)

    plsc.subcore_barrier()
    pltpu.sync_copy(acc_sh.at[pl.ds(s * STRIPE, STRIPE)],
                    out_hbm.at[c, pl.ds(s * STRIPE, STRIPE)])


def _n2n(v):
    v = jnp.where(jnp.isnan(v), 0.0, v)
    v = jnp.where(v == jnp.inf, CLIPV, v)
    v = jnp.where(v == -jnp.inf, -CLIPV, v)
    return v


def _tc_dis_body(degp_ref, o_ref):
    deg = jnp.sum(degp_ref[...], axis=0, keepdims=True)
    row = lax.broadcasted_iota(jnp.int32, (1, NP_), 1)
    deg = deg + jnp.where(row < N_NODES, 1.0, 0.0)
    o_ref[...] = jnp.where(deg > 0, lax.rsqrt(deg), 0.0)


def _tc_xws_body(x_ref, w_ref, dis_ref, o_ref):
    xw = jnp.dot(x_ref[...], w_ref[...],
                 preferred_element_type=jnp.float32,
                 precision=lax.Precision.HIGHEST)
    o_ref[...] = xw * dis_ref[...]


def _tc_mid_body(p_ref, xws_ref, dis_ref, b_ref, w_ref, o_ref):
    h = p_ref[0] + p_ref[1] + xws_ref[...] * dis_ref[...] + b_ref[...]
    h = jax.nn.relu(_n2n(h))
    xw = jnp.dot(h, w_ref[...], preferred_element_type=jnp.float32,
                 precision=lax.Precision.HIGHEST)
    o_ref[...] = xw * dis_ref[...]


def _tc_post_body(p_ref, xws_ref, dis_ref, b_ref, o_ref):
    h = p_ref[0] + p_ref[1] + xws_ref[...] * dis_ref[...] + b_ref[...]
    h = jax.nn.relu(_n2n(h))
    row = lax.broadcasted_iota(jnp.int32, (NP_, 1), 0)
    h = jnp.where(row < N_NODES, h, 0.0)
    g = jnp.sum(h, axis=0, keepdims=True) * (1.0 / N_NODES)
    o_ref[...] = _n2n(g)


def kernel(x, edge_index, edge_weight, W1, b1, W2, b2):
    src = edge_index[0].astype(jnp.int32)
    dst = edge_index[1].astype(jnp.int32)
    pad_e = EP_ - N_EDGES
    src_p = jnp.concatenate([src, jnp.full((pad_e,), NP_ - 1, jnp.int32)])
    dst_p = jnp.concatenate([dst, jnp.full((pad_e,), NP_ - 1, jnp.int32)])
    ew_p = jnp.concatenate([edge_weight.astype(jnp.float32),
                            jnp.zeros((pad_e,), jnp.float32)])
    x_p = jnp.pad(x, ((0, NP_ - N_NODES), (0, 0)))
    b1r = b1.reshape(1, FDIM)
    b2r = b2.reshape(1, FDIM)

    degp = _deg_kernel(dst_p, ew_p)

    dis2d = pl.pallas_call(
        _tc_dis_body,
        out_shape=jax.ShapeDtypeStruct((1, NP_), jnp.float32),
    )(degp)
    dis_flat = dis2d.reshape(NP_)
    dis_col = dis2d.reshape(NP_, 1)

    nrm = _nrm_kernel(dst_p, ew_p, dis_flat)

    xws1 = pl.pallas_call(
        _tc_xws_body,
        out_shape=jax.ShapeDtypeStruct((NP_, FDIM), jnp.float32),
    )(x_p, W1, dis_col)

    p1 = _agg_kernel(xws1, src_p, dst_p, nrm)

    xws2 = pl.pallas_call(
        _tc_mid_body,
        out_shape=jax.ShapeDtypeStruct((NP_, FDIM), jnp.float32),
    )(p1, xws1, dis_col, b1r, W2)

    p2 = _agg_kernel(xws2, src_p, dst_p, nrm)

    g2d = pl.pallas_call(
        _tc_post_body,
        out_shape=jax.ShapeDtypeStruct((1, FDIM), jnp.float32),
    )(p2, xws2, dis_col, b2r)

    return g2d.reshape(FDIM)


# speedup vs baseline: 29.6698x; 2.9718x over previous
import functools

import jax
import jax.numpy as jnp
from jax import lax
from jax.experimental import pallas as pl
from jax.experimental.pallas import tpu as pltpu
from jax.experimental.pallas import tpu_sc as plsc

N_NODES = 10000
N_EDGES = 320000
FDIM = 128
CLIPV = 100000.0

NP_ = 10240
EP_ = 327680
NC = 2
NS = 16
NW = NC * NS
EPW = EP_ // NW
BLK = 80
NBLK = EPW // BLK
STRIPE = NP_ // NS

_mesh = plsc.VectorSubcoreMesh(core_axis_name="c", subcore_axis_name="s",
                               num_cores=NC, num_subcores=NS)
_sc_params = pltpu.CompilerParams(needs_layout_passes=False)


@functools.partial(
    pl.kernel,
    out_type=jax.ShapeDtypeStruct((NW, NP_), jnp.float32),
    mesh=_mesh,
    scratch_types=[
        pltpu.VMEM((NP_,), jnp.float32),
        pltpu.VMEM((EPW,), jnp.int32),
        pltpu.VMEM((EPW,), jnp.float32),
    ],
    compiler_params=_sc_params,
)
def _deg_kernel(dst_hbm, ew_hbm, out_hbm, deg_v, dst_v, ew_v):
    c = lax.axis_index("c")
    s = lax.axis_index("s")
    wid = c * NS + s
    base = wid * EPW
    pltpu.sync_copy(dst_hbm.at[pl.ds(base, EPW)], dst_v)
    pltpu.sync_copy(ew_hbm.at[pl.ds(base, EPW)], ew_v)

    def zero_body(i, _):
        deg_v[pl.ds(i * 16, 16)] = jnp.zeros((16,), jnp.float32)
        return _
    lax.fori_loop(0, NP_ // 16, zero_body, None)

    def acc_body(i, _):
        idx = dst_v[pl.ds(i * 16, 16)]
        w = ew_v[pl.ds(i * 16, 16)]
        plsc.addupdate_scatter(deg_v, [idx], w)
        return _
    lax.fori_loop(0, EPW // 16, acc_body, None)

    pltpu.sync_copy(deg_v, out_hbm.at[wid])


@functools.partial(
    pl.kernel,
    out_type=jax.ShapeDtypeStruct((EP_,), jnp.float32),
    mesh=_mesh,
    scratch_types=[
        pltpu.VMEM((NP_,), jnp.float32),
        pltpu.VMEM((EPW,), jnp.int32),
        pltpu.VMEM((EPW,), jnp.float32),
        pltpu.VMEM((EPW,), jnp.float32),
    ],
    compiler_params=_sc_params,
)
def _nrm_kernel(dst_hbm, ew_hbm, dis_hbm, out_hbm, dis_v, dst_v, ew_v, nrm_v):
    c = lax.axis_index("c")
    s = lax.axis_index("s")
    wid = c * NS + s
    base = wid * EPW
    pltpu.sync_copy(dis_hbm, dis_v)
    pltpu.sync_copy(dst_hbm.at[pl.ds(base, EPW)], dst_v)
    pltpu.sync_copy(ew_hbm.at[pl.ds(base, EPW)], ew_v)

    def body(i, _):
        idx = dst_v[pl.ds(i * 16, 16)]
        disg = plsc.load_gather(dis_v, [idx])
        nrm_v[pl.ds(i * 16, 16)] = ew_v[pl.ds(i * 16, 16)] * disg
        return _
    lax.fori_loop(0, EPW // 16, body, None)
    pltpu.sync_copy(nrm_v, out_hbm.at[pl.ds(base, EPW)])


@functools.partial(
    pl.kernel,
    out_type=jax.ShapeDtypeStruct((NC, NP_, FDIM), jnp.float32),
    mesh=_mesh,
    scratch_types=(
        [pltpu.VMEM((BLK,), jnp.int32) for _ in range(8)]
        + [pltpu.VMEM((BLK,), jnp.float32) for _ in range(4)]
        + [pltpu.VMEM((BLK, FDIM), jnp.float32) for _ in range(4)]
        + [pltpu.VMEM_SHARED((NP_, FDIM), jnp.float32)]
        + [pltpu.SemaphoreType.DMA for _ in range(12)]
    ),
    compiler_params=_sc_params,
)
def _agg_kernel(tbl_hbm, src_hbm, dst_hbm, nrm_hbm, out_hbm,
                src0, src1, src2, src3, dst0, dst1, dst2, dst3,
                nr0, nr1, nr2, nr3, rows0, rows1, rows2, rows3, acc_sh,
                isem0, isem1, isem2, isem3, gsem0, gsem1, gsem2, gsem3,
                ssem0, ssem1, ssem2, ssem3):
    c = lax.axis_index("c")
    s = lax.axis_index("s")
    wid = c * NS + s
    base = wid * EPW

    idx_sets = ((src0, dst0, nr0, isem0), (src1, dst1, nr1, isem1),
                (src2, dst2, nr2, isem2), (src3, dst3, nr3, isem3))
    row_sets = ((rows0, gsem0, ssem0), (rows1, gsem1, ssem1),
                (rows2, gsem2, ssem2), (rows3, gsem3, ssem3))

    def zrow_body(i, _):
        for j in range(FDIM // 16):
            rows0[i, pl.ds(j * 16, 16)] = jnp.zeros((16,), jnp.float32)
        return _
    lax.fori_loop(0, BLK, zrow_body, None)
    for k in range(STRIPE // BLK):
        pltpu.sync_copy(rows0, acc_sh.at[pl.ds(s * STRIPE + k * BLK, BLK)])
    plsc.subcore_barrier()

    def start_idx(b, iset):
        eb = base + b * BLK
        pltpu.async_copy(src_hbm.at[pl.ds(eb, BLK)], iset[0], iset[3])
        pltpu.async_copy(dst_hbm.at[pl.ds(eb, BLK)], iset[1], iset[3])
        pltpu.async_copy(nrm_hbm.at[pl.ds(eb, BLK)], iset[2], iset[3])

    def wait_idx(iset):
        pltpu.make_async_copy(src_hbm.at[pl.ds(0, BLK)], iset[0],
                              iset[3]).wait()
        pltpu.make_async_copy(dst_hbm.at[pl.ds(0, BLK)], iset[1],
                              iset[3]).wait()
        pltpu.make_async_copy(nrm_hbm.at[pl.ds(0, BLK)], iset[2],
                              iset[3]).wait()

    def start_gather(iset, rset):
        pltpu.async_copy(tbl_hbm.at[iset[0]], rset[0], rset[1])

    def wait_gather(rset):
        pltpu.make_async_copy(tbl_hbm.at[pl.ds(0, BLK)], rset[0],
                              rset[1]).wait()

    def scale(rset, iset):
        rows_v, nrm_v = rset[0], iset[2]

        def grp_body(g, _g):
            nrm = nrm_v[pl.ds(g * 16, 16)]
            for l in range(16):
                e = g * 16 + l
                sc = lax.broadcast_in_dim(nrm[l], (16,), ())
                for j in range(FDIM // 16):
                    rows_v[e, pl.ds(j * 16, 16)] = (
                        rows_v[e, pl.ds(j * 16, 16)] * sc)
            return _g
        lax.fori_loop(0, BLK // 16, grp_body, None)

    def scatter(rset, iset):
        pltpu.async_copy(rset[0], acc_sh.at[iset[1]], rset[2], add=True)
        pltpu.make_async_copy(tbl_hbm.at[pl.ds(0, BLK)], rset[0],
                              rset[2]).wait()

    def visit(b, q, pf_idx, pf_gather):
        rset = row_sets[q]
        iset = idx_sets[q]
        wait_gather(rset)
        scale(rset, iset)
        scatter(rset, iset)
        if pf_idx:
            start_idx(b + 4, iset)
        if pf_gather:
            nq = (q + 3) % 4
            wait_idx(idx_sets[nq])
            start_gather(idx_sets[nq], row_sets[nq])

    for q in range(4):
        start_idx(q, idx_sets[q])
    for q in range(3):
        wait_idx(idx_sets[q])
        start_gather(idx_sets[q], row_sets[q])

    def quad_body(k, _):
        b = k * 4
        for q in range(4):
            visit(b + q, q, True, True)
        return _
    lax.fori_loop(0, NBLK // 4 - 1, quad_body, None)

    bq = NBLK - 4
    visit(bq + 0, 0, False, True)
    visit(bq + 1, 1, False, False)
    visit(bq + 2, 2, False, False)
    visit(bq + 3, 3, False, False)

    plsc.subcore_barrier()
    pltpu.sync_copy(acc_sh.at[pl.ds(s * STRIPE, STRIPE)],
                    out_hbm.at[c, pl.ds(s * STRIPE, STRIPE)])


def _n2n(v):
    v = jnp.where(jnp.isnan(v), 0.0, v)
    v = jnp.where(v == jnp.inf, CLIPV, v)
    v = jnp.where(v == -jnp.inf, -CLIPV, v)
    return v


def _tc_dis_body(degp_ref, o_ref):
    deg = jnp.sum(degp_ref[...], axis=0, keepdims=True)
    row = lax.broadcasted_iota(jnp.int32, (1, NP_), 1)
    deg = deg + jnp.where(row < N_NODES, 1.0, 0.0)
    o_ref[...] = jnp.where(deg > 0, lax.rsqrt(deg), 0.0)


def _tc_xws_body(x_ref, w_ref, dis_ref, o_ref):
    xw = jnp.dot(x_ref[...], w_ref[...],
                 preferred_element_type=jnp.float32,
                 precision=lax.Precision.HIGHEST)
    o_ref[...] = xw * dis_ref[...]


def _tc_mid_body(p_ref, xws_ref, dis_ref, b_ref, w_ref, o_ref):
    h = p_ref[0] + p_ref[1] + xws_ref[...] * dis_ref[...] + b_ref[...]
    h = jax.nn.relu(_n2n(h))
    xw = jnp.dot(h, w_ref[...], preferred_element_type=jnp.float32,
                 precision=lax.Precision.HIGHEST)
    o_ref[...] = xw * dis_ref[...]


def _tc_post_body(p_ref, xws_ref, dis_ref, b_ref, o_ref):
    h = p_ref[0] + p_ref[1] + xws_ref[...] * dis_ref[...] + b_ref[...]
    h = jax.nn.relu(_n2n(h))
    row = lax.broadcasted_iota(jnp.int32, (NP_, 1), 0)
    h = jnp.where(row < N_NODES, h, 0.0)
    g = jnp.sum(h, axis=0, keepdims=True) * (1.0 / N_NODES)
    o_ref[...] = _n2n(g)


def kernel(x, edge_index, edge_weight, W1, b1, W2, b2):
    src = edge_index[0].astype(jnp.int32)
    dst = edge_index[1].astype(jnp.int32)
    pad_e = EP_ - N_EDGES
    pad_idx = N_NODES + jnp.arange(pad_e, dtype=jnp.int32) % (NP_ - N_NODES)
    src_p = jnp.concatenate([src, pad_idx])
    dst_p = jnp.concatenate([dst, pad_idx])
    ew_p = jnp.concatenate([edge_weight.astype(jnp.float32),
                            jnp.zeros((pad_e,), jnp.float32)])
    x_p = jnp.pad(x, ((0, NP_ - N_NODES), (0, 0)))
    b1r = b1.reshape(1, FDIM)
    b2r = b2.reshape(1, FDIM)

    degp = _deg_kernel(dst_p, ew_p)

    dis2d = pl.pallas_call(
        _tc_dis_body,
        out_shape=jax.ShapeDtypeStruct((1, NP_), jnp.float32),
    )(degp)
    dis_flat = dis2d.reshape(NP_)
    dis_col = dis2d.reshape(NP_, 1)

    nrm = _nrm_kernel(dst_p, ew_p, dis_flat)

    xws1 = pl.pallas_call(
        _tc_xws_body,
        out_shape=jax.ShapeDtypeStruct((NP_, FDIM), jnp.float32),
    )(x_p, W1, dis_col)

    p1 = _agg_kernel(xws1, src_p, dst_p, nrm)

    xws2 = pl.pallas_call(
        _tc_mid_body,
        out_shape=jax.ShapeDtypeStruct((NP_, FDIM), jnp.float32),
    )(p1, xws1, dis_col, b1r, W2)

    p2 = _agg_kernel(xws2, src_p, dst_p, nrm)

    g2d = pl.pallas_call(
        _tc_post_body,
        out_shape=jax.ShapeDtypeStruct((1, FDIM), jnp.float32),
    )(p2, xws2, dis_col, b2r)

    return g2d.reshape(FDIM)


# speedup vs baseline: 29.9041x; 1.0079x over previous
import functools

import jax
import jax.numpy as jnp
from jax import lax
from jax.experimental import pallas as pl
from jax.experimental.pallas import tpu as pltpu
from jax.experimental.pallas import tpu_sc as plsc

N_NODES = 10000
N_EDGES = 320000
FDIM = 128
CLIPV = 100000.0

NP_ = 10240
EP_ = 327680
NC = 2
NS = 16
NW = NC * NS
EPW = EP_ // NW
BLK = 80
NBLK = EPW // BLK
STRIPE = NP_ // NS

_mesh = plsc.VectorSubcoreMesh(core_axis_name="c", subcore_axis_name="s",
                               num_cores=NC, num_subcores=NS)
_sc_params = pltpu.CompilerParams(needs_layout_passes=False)


@functools.partial(
    pl.kernel,
    out_type=jax.ShapeDtypeStruct((NW, NP_), jnp.float32),
    mesh=_mesh,
    scratch_types=[
        pltpu.VMEM((NP_,), jnp.float32),
        pltpu.VMEM((EPW,), jnp.int32),
        pltpu.VMEM((EPW,), jnp.float32),
    ],
    compiler_params=_sc_params,
)
def _deg_kernel(dst_hbm, ew_hbm, out_hbm, deg_v, dst_v, ew_v):
    c = lax.axis_index("c")
    s = lax.axis_index("s")
    wid = c * NS + s
    base = wid * EPW
    pltpu.sync_copy(dst_hbm.at[pl.ds(base, EPW)], dst_v)
    pltpu.sync_copy(ew_hbm.at[pl.ds(base, EPW)], ew_v)

    def zero_body(i, _):
        deg_v[pl.ds(i * 16, 16)] = jnp.zeros((16,), jnp.float32)
        return _
    lax.fori_loop(0, NP_ // 16, zero_body, None)

    def acc_body(i, _):
        idx = dst_v[pl.ds(i * 16, 16)]
        w = ew_v[pl.ds(i * 16, 16)]
        plsc.addupdate_scatter(deg_v, [idx], w)
        return _
    lax.fori_loop(0, EPW // 16, acc_body, None)

    pltpu.sync_copy(deg_v, out_hbm.at[wid])


@functools.partial(
    pl.kernel,
    out_type=jax.ShapeDtypeStruct((EP_,), jnp.float32),
    mesh=_mesh,
    scratch_types=[
        pltpu.VMEM((NP_,), jnp.float32),
        pltpu.VMEM((EPW,), jnp.int32),
        pltpu.VMEM((EPW,), jnp.float32),
        pltpu.VMEM((EPW,), jnp.float32),
    ],
    compiler_params=_sc_params,
)
def _nrm_kernel(dst_hbm, ew_hbm, dis_hbm, out_hbm, dis_v, dst_v, ew_v, nrm_v):
    c = lax.axis_index("c")
    s = lax.axis_index("s")
    wid = c * NS + s
    base = wid * EPW
    pltpu.sync_copy(dis_hbm, dis_v)
    pltpu.sync_copy(dst_hbm.at[pl.ds(base, EPW)], dst_v)
    pltpu.sync_copy(ew_hbm.at[pl.ds(base, EPW)], ew_v)

    def body(i, _):
        idx = dst_v[pl.ds(i * 16, 16)]
        disg = plsc.load_gather(dis_v, [idx])
        nrm_v[pl.ds(i * 16, 16)] = ew_v[pl.ds(i * 16, 16)] * disg
        return _
    lax.fori_loop(0, EPW // 16, body, None)
    pltpu.sync_copy(nrm_v, out_hbm.at[pl.ds(base, EPW)])


@functools.partial(
    pl.kernel,
    out_type=jax.ShapeDtypeStruct((NC, NP_, FDIM), jnp.float32),
    mesh=_mesh,
    scratch_types=(
        [pltpu.VMEM((BLK,), jnp.int32) for _ in range(8)]
        + [pltpu.VMEM((BLK,), jnp.float32) for _ in range(4)]
        + [pltpu.VMEM((BLK, FDIM), jnp.float32) for _ in range(4)]
        + [pltpu.VMEM_SHARED((NP_, FDIM), jnp.float32)]
        + [pltpu.SemaphoreType.DMA for _ in range(12)]
    ),
    compiler_params=_sc_params,
)
def _agg_kernel(tbl_hbm, src_hbm, dst_hbm, nrm_hbm, out_hbm,
                src0, src1, src2, src3, dst0, dst1, dst2, dst3,
                nr0, nr1, nr2, nr3, rows0, rows1, rows2, rows3, acc_sh,
                isem0, isem1, isem2, isem3, gsem0, gsem1, gsem2, gsem3,
                ssem0, ssem1, ssem2, ssem3):
    c = lax.axis_index("c")
    s = lax.axis_index("s")
    wid = c * NS + s
    base = wid * EPW

    idx_sets = ((src0, dst0, nr0, isem0), (src1, dst1, nr1, isem1),
                (src2, dst2, nr2, isem2), (src3, dst3, nr3, isem3))
    row_sets = ((rows0, gsem0, ssem0), (rows1, gsem1, ssem1),
                (rows2, gsem2, ssem2), (rows3, gsem3, ssem3))

    def zrow_body(i, _):
        for j in range(FDIM // 16):
            rows0[i, pl.ds(j * 16, 16)] = jnp.zeros((16,), jnp.float32)
        return _
    lax.fori_loop(0, BLK, zrow_body, None)
    for k in range(STRIPE // BLK):
        pltpu.sync_copy(rows0, acc_sh.at[pl.ds(s * STRIPE + k * BLK, BLK)])
    plsc.subcore_barrier()

    def start_idx(b, iset):
        eb = base + b * BLK
        pltpu.async_copy(src_hbm.at[pl.ds(eb, BLK)], iset[0], iset[3])
        pltpu.async_copy(dst_hbm.at[pl.ds(eb, BLK)], iset[1], iset[3])
        pltpu.async_copy(nrm_hbm.at[pl.ds(eb, BLK)], iset[2], iset[3])

    def wait_idx(iset):
        pltpu.make_async_copy(src_hbm.at[pl.ds(0, BLK)], iset[0],
                              iset[3]).wait()
        pltpu.make_async_copy(dst_hbm.at[pl.ds(0, BLK)], iset[1],
                              iset[3]).wait()
        pltpu.make_async_copy(nrm_hbm.at[pl.ds(0, BLK)], iset[2],
                              iset[3]).wait()

    def start_gather(iset, rset):
        pltpu.async_copy(tbl_hbm.at[iset[0]], rset[0], rset[1])

    def wait_gather(rset):
        pltpu.make_async_copy(tbl_hbm.at[pl.ds(0, BLK)], rset[0],
                              rset[1]).wait()

    def scale(rset, iset):
        rows_v, nrm_v = rset[0], iset[2]

        def grp_body(g, _g):
            nrm = nrm_v[pl.ds(g * 16, 16)]
            for l in range(16):
                e = g * 16 + l
                sc = lax.broadcast_in_dim(nrm[l], (16,), ())
                for j in range(FDIM // 16):
                    rows_v[e, pl.ds(j * 16, 16)] = (
                        rows_v[e, pl.ds(j * 16, 16)] * sc)
            return _g
        lax.fori_loop(0, BLK // 16, grp_body, None)

    def scatter_issue(rset, iset):
        pltpu.async_copy(rset[0], acc_sh.at[iset[1]], rset[2], add=True)

    def scatter_wait(rset):
        pltpu.make_async_copy(tbl_hbm.at[pl.ds(0, BLK)], rset[0],
                              rset[2]).wait()

    def visit(b, q, pf_idx, pf_gather):
        rset = row_sets[q]
        iset = idx_sets[q]
        wait_gather(rset)
        scale(rset, iset)
        scatter_issue(rset, iset)
        if pf_idx:
            start_idx(b + 4, iset)
        if pf_gather:
            nq = (q + 3) % 4
            wait_idx(idx_sets[nq])
            start_gather(idx_sets[nq], row_sets[nq])
        scatter_wait(rset)

    for q in range(4):
        start_idx(q, idx_sets[q])
    for q in range(3):
        wait_idx(idx_sets[q])
        start_gather(idx_sets[q], row_sets[q])

    def quad_body(k, _):
        b = k * 4
        for q in range(4):
            visit(b + q, q, True, True)
        return _
    lax.fori_loop(0, NBLK // 4 - 1, quad_body, None)

    bq = NBLK - 4
    visit(bq + 0, 0, False, True)
    visit(bq + 1, 1, False, False)
    visit(bq + 2, 2, False, False)
    visit(bq + 3, 3, False, False)

    plsc.subcore_barrier()
    pltpu.sync_copy(acc_sh.at[pl.ds(s * STRIPE, STRIPE)],
                    out_hbm.at[c, pl.ds(s * STRIPE, STRIPE)])


def _n2n(v):
    v = jnp.where(jnp.isnan(v), 0.0, v)
    v = jnp.where(v == jnp.inf, CLIPV, v)
    v = jnp.where(v == -jnp.inf, -CLIPV, v)
    return v


def _tc_dis_body(degp_ref, o_ref):
    deg = jnp.sum(degp_ref[...], axis=0, keepdims=True)
    row = lax.broadcasted_iota(jnp.int32, (1, NP_), 1)
    deg = deg + jnp.where(row < N_NODES, 1.0, 0.0)
    o_ref[...] = jnp.where(deg > 0, lax.rsqrt(deg), 0.0)


def _tc_xws_body(x_ref, w_ref, dis_ref, o_ref):
    xw = jnp.dot(x_ref[...], w_ref[...],
                 preferred_element_type=jnp.float32,
                 precision=lax.Precision.HIGHEST)
    o_ref[...] = xw * dis_ref[...]


def _tc_mid_body(p_ref, xws_ref, dis_ref, b_ref, w_ref, o_ref):
    h = p_ref[0] + p_ref[1] + xws_ref[...] * dis_ref[...] + b_ref[...]
    h = jax.nn.relu(_n2n(h))
    xw = jnp.dot(h, w_ref[...], preferred_element_type=jnp.float32,
                 precision=lax.Precision.HIGHEST)
    o_ref[...] = xw * dis_ref[...]


def _tc_post_body(p_ref, xws_ref, dis_ref, b_ref, o_ref):
    h = p_ref[0] + p_ref[1] + xws_ref[...] * dis_ref[...] + b_ref[...]
    h = jax.nn.relu(_n2n(h))
    row = lax.broadcasted_iota(jnp.int32, (NP_, 1), 0)
    h = jnp.where(row < N_NODES, h, 0.0)
    g = jnp.sum(h, axis=0, keepdims=True) * (1.0 / N_NODES)
    o_ref[...] = _n2n(g)


def kernel(x, edge_index, edge_weight, W1, b1, W2, b2):
    src = edge_index[0].astype(jnp.int32)
    dst = edge_index[1].astype(jnp.int32)
    pad_e = EP_ - N_EDGES
    pad_idx = N_NODES + jnp.arange(pad_e, dtype=jnp.int32) % (NP_ - N_NODES)
    src_p = jnp.concatenate([src, pad_idx])
    dst_p = jnp.concatenate([dst, pad_idx])
    ew_p = jnp.concatenate([edge_weight.astype(jnp.float32),
                            jnp.zeros((pad_e,), jnp.float32)])
    x_p = jnp.pad(x, ((0, NP_ - N_NODES), (0, 0)))
    b1r = b1.reshape(1, FDIM)
    b2r = b2.reshape(1, FDIM)

    degp = _deg_kernel(dst_p, ew_p)

    dis2d = pl.pallas_call(
        _tc_dis_body,
        out_shape=jax.ShapeDtypeStruct((1, NP_), jnp.float32),
    )(degp)
    dis_flat = dis2d.reshape(NP_)
    dis_col = dis2d.reshape(NP_, 1)

    nrm = _nrm_kernel(dst_p, ew_p, dis_flat)

    xws1 = pl.pallas_call(
        _tc_xws_body,
        out_shape=jax.ShapeDtypeStruct((NP_, FDIM), jnp.float32),
    )(x_p, W1, dis_col)

    p1 = _agg_kernel(xws1, src_p, dst_p, nrm)

    xws2 = pl.pallas_call(
        _tc_mid_body,
        out_shape=jax.ShapeDtypeStruct((NP_, FDIM), jnp.float32),
    )(p1, xws1, dis_col, b1r, W2)

    p2 = _agg_kernel(xws2, src_p, dst_p, nrm)

    g2d = pl.pallas_call(
        _tc_post_body,
        out_shape=jax.ShapeDtypeStruct((1, FDIM), jnp.float32),
    )(p2, xws2, dis_col, b2r)

    return g2d.reshape(FDIM)
